# baseline (speedup 1.0000x reference)
import functools

import jax
import jax.numpy as jnp
import numpy as np
from jax import lax
from jax.experimental import pallas as pl
from jax.experimental.pallas import tpu as pltpu
from jax.experimental.pallas import tpu_sc as plsc

HIDDEN = 128
HH = 64
MP = 5

_NC = 2
_NS = 16
_NW = _NC * _NS
_CH = 128

_BE = 2000
_BN = 2000


def _dot(x, w):
    return jnp.dot(x.astype(jnp.bfloat16), w.astype(jnp.bfloat16),
                   preferred_element_type=jnp.float32)


def _dotf(x, w):
    return jnp.dot(x, w, preferred_element_type=jnp.float32)


def _pk(a, b):
    ua = lax.bitcast_convert_type(a.astype(jnp.bfloat16),
                                  jnp.uint16).astype(jnp.uint32)
    ub = lax.bitcast_convert_type(b.astype(jnp.bfloat16),
                                  jnp.uint16).astype(jnp.uint32)
    return lax.bitcast_convert_type(ua | (ub << 16), jnp.int32)


def _unpk_lo(p):
    return lax.bitcast_convert_type(p << 16, jnp.float32)


def _unpk_hi(p):
    return lax.bitcast_convert_type(p & jnp.int32(-65536), jnp.float32)


def _mlp_tail(x1, w2, b2, w3, b3, g, bt):
    x = jnp.tanh(x1)
    x = jnp.tanh(_dot(x, w2[...]) + b2[...])
    x = _dot(x, w3[...]) + b3[...]
    mu = jnp.mean(x, axis=-1, keepdims=True)
    xc = x - mu
    var = jnp.mean(xc * xc, axis=-1, keepdims=True)
    return xc * lax.rsqrt(var + 1e-5) * g[...] + bt[...]


def _enc_e_body(e_ref, w1, b1, w2, b2, w3, b3, g, bt, out_ref):
    x1 = _dot(e_ref[...], w1[...]) + b1[...]
    out_ref[...] = _mlp_tail(x1, w2, b2, w3, b3, g, bt)


def _enc_vh_body(v_ref, h_ref, ca_ref, lrow, w1, b1, w2, b2, w3, b3, g, bt,
                 v_out, h_out, vhpk_out):
    x1 = _dot(v_ref[...], w1[...]) + b1[...]
    vv = _mlp_tail(x1, w2, b2, w3, b3, g, bt)
    v_out[...] = vv
    hh = (h_ref[...] * ca_ref[...]) * lrow[...]
    h_out[...] = hh
    hp = _pk(hh[:, :HH // 2], hh[:, HH // 2:HH])
    pkv = _pk(vv[:, :HIDDEN // 2], vv[:, HIDDEN // 2:])
    vhpk_out[...] = jnp.concatenate([pkv, hp, jnp.zeros_like(hp)], axis=-1)


def _edge_body(vhs, vhd, e, w1sa, w1sb, w1da, w1db, w1e, b1,
               w2, b2, w3, b3, g, bt, out):
    H2 = HIDDEN // 2
    ps = vhs[...][:, :H2]
    pd = vhd[...][:, :H2]
    x1 = (_dot(_unpk_lo(ps), w1sa[...]) + _dot(_unpk_hi(ps), w1sb[...])
          + _dot(_unpk_lo(pd), w1da[...]) + _dot(_unpk_hi(pd), w1db[...])
          + _dot(e[...], w1e[...]) + b1[...])
    out[...] = e[...] + _mlp_tail(x1, w2, b2, w3, b3, g, bt)


def _node_body(v, p, h_old, php, w1v, w1p, b1, w2, b2, w3, b3, g, bt,
               out, pk_out, h_out):
    pp = p[...]
    v1 = pp[0, 0] + pp[1, 0]
    x1 = _dot(v[...], w1v[...]) + _dot(v1, w1p[...]) + b1[...]
    vv = v[...] + _mlp_tail(x1, w2, b2, w3, b3, g, bt)
    out[...] = vv
    hq = php[...]
    hh = h_old[...] + hq[0] - hq[1]
    h_out[...] = hh
    pkv = _pk(vv[:, :HIDDEN // 2], vv[:, HIDDEN // 2:])
    hp = _pk(hh[:, :HH // 2], hh[:, HH // 2:HH])
    pk_out[...] = jnp.concatenate(
        [pkv, hp, jnp.zeros_like(hp)], axis=-1)


def _flux_body(vhs, vhd, m, nx, ny, el,
               w1ha, w1hb, w1m, w1va, w1vb, w1da, w1db,
               b1, w2, b2, w3, b3, g, bt, se, so,
               m_out, q_out):
    ps = vhs[...]
    pd = vhd[...]
    H2 = HIDDEN // 2
    Q = HH // 2
    vsp, vdp = ps[:, :H2], pd[:, :H2]
    hsp, hdp = ps[:, H2:H2 + Q], pd[:, H2:H2 + Q]
    hsum_lo = _unpk_lo(hsp) + _unpk_lo(hdp)
    hsum_hi = _unpk_hi(hsp) + _unpk_hi(hdp)
    x1 = (_dot(hsum_lo, w1ha[...]) + _dot(hsum_hi, w1hb[...])
          + _dot(m[...], w1m[...])
          + _dot(_unpk_lo(vsp), w1va[...]) + _dot(_unpk_hi(vsp), w1vb[...])
          + _dot(_unpk_lo(vdp), w1da[...]) + _dot(_unpk_hi(vdp), w1db[...])
          + b1[...])
    mn = m[...] + _mlp_tail(x1, w2, b2, w3, b3, g, bt)
    me = _dotf(mn, se[...])
    mo = _dotf(mn, so[...])
    m_out[...] = mn
    q = (me * nx[...] + mo * ny[...]) * el[...]
    q_out[...] = jnp.concatenate([q, jnp.zeros_like(q)], axis=-1)


def _dec_body(h, php, lrow, ca, out):
    lv = lrow[...]
    hq = php[...]
    hh = h[...] + hq[0] - hq[1]
    s = jnp.sum(lv * lv)
    out[...] = jnp.sum(hh * lv, axis=-1, keepdims=True) / s / ca[...]


def _rows(B, D):
    return pl.BlockSpec((B, D), lambda i: (i, 0))


def _full(a):
    nd = a.ndim
    return pl.BlockSpec(a.shape, lambda i: (0,) * nd)


def _tc_call(body, grid, in_specs, out_specs, out_shape, args):
    return pl.pallas_call(
        body, grid=grid, in_specs=in_specs, out_specs=out_specs,
        out_shape=out_shape)(*args)


def _sc_gather(tables, pack):
    main_src, tail_src, main_dst, tail_dst = pack
    n_main, ch = main_src.shape[1], main_src.shape[2]
    tail = tail_src.shape[1]
    per_w = n_main * ch + tail
    E = per_w * _NW
    nt = len(tables)
    ns = 2 * nt
    Ds = [int(t.shape[1]) for t in tables]
    dts = [t.dtype for t in tables]
    nb = 3
    assert n_main % nb == 0
    mesh = plsc.VectorSubcoreMesh(core_axis_name="c", subcore_axis_name="s")
    out_type = [jax.ShapeDtypeStruct((E, D), dt)
                for D, dt in zip(Ds, dts) for _ in range(2)]
    scratch = [pltpu.VMEM((n_main, ch), jnp.int32),
               pltpu.VMEM((n_main, ch), jnp.int32),
               pltpu.VMEM((tail,), jnp.int32),
               pltpu.VMEM((tail,), jnp.int32)]
    for D, dt in zip(Ds, dts):
        for _ in range(2):
            for _ in range(nb):
                scratch.append(pltpu.VMEM((ch, D), dt))
            scratch.append(pltpu.VMEM((tail, D), dt))
    nsem = ns * nb * 2 + 1
    scratch += [pltpu.SemaphoreType.DMA] * nsem

    def body(*refs):
        tab = refs[:nt]
        im = (refs[nt], refs[nt + 2])
        it = (refs[nt + 1], refs[nt + 3])
        outs = refs[nt + 4:nt + 4 + ns]
        scr = refs[nt + 4 + ns:]
        idx_all = scr[0:2]
        idx_tl = scr[2:4]
        rows = [[scr[4 + s * (nb + 1) + b] for b in range(nb)]
                for s in range(ns)]
        rtail = [scr[4 + s * (nb + 1) + nb] for s in range(ns)]
        sems = scr[4 + ns * (nb + 1):]
        sem_g = [[sems[s * nb + b] for b in range(nb)] for s in range(ns)]
        sem_w = [[sems[ns * nb + s * nb + b] for b in range(nb)]
                 for s in range(ns)]
        sem_x = sems[-1]
        wid = lax.axis_index("s") * _NC + lax.axis_index("c")
        base_w = wid * per_w

        for k in range(2):
            pltpu.sync_copy(im[k].at[wid], idx_all[k])
            pltpu.sync_copy(it[k].at[wid], idx_tl[k])

        def g_desc(s, b, j):
            t, k = s // 2, s % 2
            return pltpu.make_async_copy(
                tab[t].at[idx_all[k].at[j]], rows[s][b], sem_g[s][b])

        def w_desc(s, b, j):
            t, k = s // 2, s % 2
            return pltpu.make_async_copy(
                rows[s][b], outs[2 * t + k].at[pl.ds(base_w + j * ch, ch)],
                sem_w[s][b])

        for b in range(min(nb - 1, n_main)):
            for s in range(ns):
                g_desc(s, b, b).start()

        @pl.loop(0, n_main, step=nb)
        def _main(j0):
            for db in range(nb):
                j = j0 + db
                bp = (db - 1) % nb
                for s in range(ns):
                    @pl.when(j >= 1)
                    def _wb_done():
                        w_desc(s, bp, j - 1).wait()

                    @pl.when(j + nb - 1 < n_main)
                    def _next_g():
                        g_desc(s, bp, j + nb - 1).start()

                    g_desc(s, db, j).wait()
                    w_desc(s, db, j).start()

        for s in range(ns):
            w_desc(s, (n_main - 1) % nb, n_main - 1).wait()

        if tail:
            for s in range(ns):
                t, k = s // 2, s % 2
                pltpu.make_async_copy(
                    tab[t].at[idx_tl[k]], rtail[s], sem_x).start()
            for s in range(ns):
                t, k = s // 2, s % 2
                pltpu.make_async_copy(
                    tab[t].at[idx_tl[k]], rtail[s], sem_x).wait()
                pltpu.sync_copy(
                    rtail[s],
                    outs[2 * t + k].at[pl.ds(base_w + n_main * ch, tail)])

    f = pl.kernel(body, out_type=out_type, mesh=mesh, scratch_types=scratch)
    return f(*tables, main_src, tail_src, main_dst, tail_dst)


def _sc_scatter_split(data, main_comb, tail_comb, zrows, n_nodes):
    E, D = data.shape
    n_main, ch = main_comb.shape[1], main_comb.shape[2]
    tail = tail_comb.shape[1]
    per_c = n_main * ch + tail
    rpt = (-(-n_nodes // _NS) + 7) // 8 * 8
    last = n_nodes - (_NS - 1) * rpt
    mesh = plsc.VectorSubcoreMesh(core_axis_name="c", subcore_axis_name="s")
    out_type = jax.ShapeDtypeStruct((2, n_nodes, D), jnp.float32)
    nb = 2
    assert n_main % nb == 0
    scratch = ([pltpu.VMEM_SHARED((n_nodes, D), jnp.float32)]
               + [pltpu.VMEM((n_main, ch), jnp.int32)]
               + [pltpu.VMEM((tail,), jnp.int32)]
               + [pltpu.VMEM((ch, D), jnp.float32)] * nb
               + [pltpu.VMEM((tail, D), jnp.float32)]
               + [pltpu.SemaphoreType.DMA] * (2 * nb + 1))

    def body(d_hbm, imc, itc, zr, out, *scr):
        accum = scr[0]
        idx_all = scr[1]
        idx_tl = scr[2]
        rows = scr[3:3 + nb]
        rtail = scr[3 + nb]
        sems = scr[4 + nb:]
        sem_l = sems[0:nb]
        sem_s = sems[nb:2 * nb]
        sem_x = sems[-1]
        cid = lax.axis_index("c")
        sid = lax.axis_index("s")
        base_w = sid * per_c

        @pl.when(sid < _NS - 1)
        def _zmain():
            pltpu.sync_copy(zr, accum.at[pl.ds(sid * rpt, rpt)])

        @pl.when(sid == _NS - 1)
        def _zlast():
            pltpu.sync_copy(zr.at[pl.ds(0, last)],
                            accum.at[pl.ds((_NS - 1) * rpt, last)])

        pltpu.sync_copy(imc.at[cid * _NS + sid], idx_all)
        pltpu.sync_copy(itc.at[cid * _NS + sid], idx_tl)
        plsc.subcore_barrier()

        def l_desc(b, j):
            return pltpu.make_async_copy(
                d_hbm.at[pl.ds(base_w + j * ch, ch)], rows[b], sem_l[b])

        def s_desc(b, j):
            return pltpu.make_async_copy(
                rows[b], accum.at[idx_all.at[j]], sem_s[b])

        for b in range(min(nb - 1, n_main)):
            l_desc(b, b).start()

        @pl.loop(0, n_main, step=nb)
        def _main(j0):
            for db in range(nb):
                j = j0 + db
                bp = (db - 1) % nb

                @pl.when(j >= 1)
                def _prev_done():
                    s_desc(bp, j - 1).wait()

                @pl.when(j + nb - 1 < n_main)
                def _next_l():
                    l_desc(bp, j + nb - 1).start()

                l_desc(db, j).wait()
                pltpu.async_copy(rows[db], accum.at[idx_all.at[j]],
                                 sem_s[db], add=True)

        s_desc((n_main - 1) % nb, n_main - 1).wait()

        if tail:
            pltpu.make_async_copy(
                d_hbm.at[pl.ds(base_w + n_main * ch, tail)], rtail,
                sem_x).start()
            pltpu.make_async_copy(
                d_hbm.at[pl.ds(base_w + n_main * ch, tail)], rtail,
                sem_x).wait()
            pltpu.sync_copy(rtail, accum.at[idx_tl], add=True)

        plsc.subcore_barrier()

        @pl.when(sid < _NS - 1)
        def _omain():
            pltpu.sync_copy(accum.at[pl.ds(sid * rpt, rpt)],
                            out.at[cid, pl.ds(sid * rpt, rpt)])

        @pl.when(sid == _NS - 1)
        def _olast():
            pltpu.sync_copy(accum.at[pl.ds((_NS - 1) * rpt, last)],
                            out.at[cid, pl.ds((_NS - 1) * rpt, last)])

    f = pl.kernel(body, out_type=out_type, mesh=mesh, scratch_types=scratch)
    return f(data, main_comb, tail_comb, zrows)


def _sc_scatter(data, pack, zrows, n_nodes, na):
    main_src, tail_src, main_dst, tail_dst = pack
    E, D = data.shape
    n_main, ch = main_src.shape[1], main_src.shape[2]
    tail = tail_src.shape[1]
    per_w = n_main * ch + tail
    rpt = (-(-n_nodes // _NS) + 7) // 8 * 8
    last = n_nodes - (_NS - 1) * rpt
    mesh = plsc.VectorSubcoreMesh(core_axis_name="c", subcore_axis_name="s")
    out_type = jax.ShapeDtypeStruct((2, na, n_nodes, D), jnp.float32)
    nb = 2
    assert n_main % nb == 0
    scratch = ([pltpu.VMEM_SHARED((n_nodes, D), jnp.float32)] * na
               + [pltpu.VMEM((n_main, ch), jnp.int32)] * 2
               + [pltpu.VMEM((tail,), jnp.int32)] * 2
               + [pltpu.VMEM((ch, D), jnp.float32)] * nb
               + [pltpu.VMEM((tail, D), jnp.float32)]
               + [pltpu.SemaphoreType.DMA] * (3 * nb + 1))

    def body(d_hbm, ims, its, imd, itd, zr, out, *scr):
        accums = scr[:na]
        idx_all = scr[na:na + 2]
        idx_tl = scr[na + 2:na + 4]
        rows = scr[na + 4:na + 4 + nb]
        rtail = scr[na + 4 + nb]
        sems = scr[na + 5 + nb:]
        sem_l = sems[0:nb]
        sem_s = [sems[nb + k * nb:nb + (k + 1) * nb]
                 for k in range(2)]
        sem_x = sems[-1]
        cid = lax.axis_index("c")
        sid = lax.axis_index("s")
        wid = sid * _NC + cid
        base_w = wid * per_w
        acc = [accums[0], accums[na - 1]]

        for a in range(na):
            @pl.when(sid < _NS - 1)
            def _zmain():
                pltpu.sync_copy(zr, accums[a].at[pl.ds(sid * rpt, rpt)])

            @pl.when(sid == _NS - 1)
            def _zlast():
                pltpu.sync_copy(zr.at[pl.ds(0, last)],
                                accums[a].at[pl.ds((_NS - 1) * rpt, last)])

        pltpu.sync_copy(ims.at[wid], idx_all[0])
        pltpu.sync_copy(imd.at[wid], idx_all[1])
        pltpu.sync_copy(its.at[wid], idx_tl[0])
        pltpu.sync_copy(itd.at[wid], idx_tl[1])
        plsc.subcore_barrier()

        def l_desc(b, j):
            return pltpu.make_async_copy(
                d_hbm.at[pl.ds(base_w + j * ch, ch)], rows[b], sem_l[b])

        def s_desc(k, b, j):
            return pltpu.make_async_copy(
                rows[b], acc[k].at[idx_all[k].at[j]], sem_s[k][b])

        for b in range(min(nb - 1, n_main)):
            l_desc(b, b).start()

        @pl.loop(0, n_main, step=nb)
        def _main(j0):
            for db in range(nb):
                j = j0 + db
                bp = (db - 1) % nb

                @pl.when(j >= 1)
                def _prev_done():
                    for k in range(2):
                        s_desc(k, bp, j - 1).wait()

                @pl.when(j + nb - 1 < n_main)
                def _next_l():
                    l_desc(bp, j + nb - 1).start()

                l_desc(db, j).wait()
                for k in range(2):
                    pltpu.async_copy(rows[db], acc[k].at[idx_all[k].at[j]],
                                     sem_s[k][db], add=True)

        for k in range(2):
            s_desc(k, (n_main - 1) % nb, n_main - 1).wait()

        if tail:
            pltpu.make_async_copy(
                d_hbm.at[pl.ds(base_w + n_main * ch, tail)], rtail,
                sem_x).start()
            pltpu.make_async_copy(
                d_hbm.at[pl.ds(base_w + n_main * ch, tail)], rtail,
                sem_x).wait()
            for k in range(2):
                pltpu.sync_copy(rtail, acc[k].at[idx_tl[k]], add=True)

        plsc.subcore_barrier()
        for a in range(na):
            @pl.when(sid < _NS - 1)
            def _omain():
                pltpu.sync_copy(accums[a].at[pl.ds(sid * rpt, rpt)],
                                out.at[cid, a, pl.ds(sid * rpt, rpt)])

            @pl.when(sid == _NS - 1)
            def _olast():
                pltpu.sync_copy(accums[a].at[pl.ds((_NS - 1) * rpt, last)],
                                out.at[cid, a, pl.ds((_NS - 1) * rpt, last)])

    f = pl.kernel(body, out_type=out_type, mesh=mesh, scratch_types=scratch)
    return f(data, main_src, tail_src, main_dst, tail_dst, zrows)


def _mlp_w(p):
    return dict(
        w1=p["l1"]["w"].T, b1=p["l1"]["b"].reshape(1, -1),
        w2=p["l2"]["w"].T, b2=p["l2"]["b"].reshape(1, -1),
        w3=p["l3"]["w"].T, b3=p["l3"]["b"].reshape(1, -1),
        g=p["g"].reshape(1, -1), bt=p["bt"].reshape(1, -1))


def kernel(h, e, v, edges, cell_areas, edge_lens, edge_normals, params):
    N = h.shape[0]
    E = e.shape[0]
    f32 = jnp.float32
    src = edges[:, 0].astype(jnp.int32)
    dst = edges[:, 1].astype(jnp.int32)
    ca = cell_areas.reshape(N, 1).astype(f32)
    nx = edge_normals[:, 0:1]
    ny = edge_normals[:, 1:2]
    el = edge_lens.reshape(E, 1)
    lrow = params["L"].reshape(1, HH)
    lrow128 = jnp.concatenate([lrow, jnp.zeros((1, HH), f32)], axis=-1)

    ge = (E // _BE,)
    gn = (N // _BN,)

    se_np = np.zeros((2 * HH, HH), np.float32)
    so_np = np.zeros((2 * HH, HH), np.float32)
    se_np[2 * np.arange(HH), np.arange(HH)] = 1.0
    so_np[2 * np.arange(HH) + 1, np.arange(HH)] = 1.0
    se = jnp.asarray(se_np)
    so = jnp.asarray(so_np)

    we = _mlp_w(params["e_enc"])
    wv = _mlp_w(params["v_enc"])

    def wlist(w, first_splits=None):
        if first_splits is None:
            firsts = [w["w1"]]
        else:
            firsts = []
            o = 0
            for sz in first_splits:
                firsts.append(w["w1"][o:o + sz])
                o += sz
        return firsts + [w["b1"], w["w2"], w["b2"], w["w3"], w["b3"],
                         w["g"], w["bt"]]

    e_h = _tc_call(
        _enc_e_body, ge,
        [_rows(_BE, e.shape[1])] + [_full(a) for a in wlist(we)],
        _rows(_BE, HIDDEN), jax.ShapeDtypeStruct((E, HIDDEN), f32),
        [e] + wlist(we))

    i32 = jnp.int32
    v_h, h_h, vh_pk = _tc_call(
        _enc_vh_body, gn,
        [_rows(_BN, v.shape[1]), _rows(_BN, 1), _rows(_BN, 1), _full(lrow128)]
        + [_full(a) for a in wlist(wv)],
        [_rows(_BN, HIDDEN), _rows(_BN, HIDDEN), _rows(_BN, HIDDEN)],
        [jax.ShapeDtypeStruct((N, HIDDEN), f32),
         jax.ShapeDtypeStruct((N, HIDDEN), f32),
         jax.ShapeDtypeStruct((N, HIDDEN), i32)],
        [v, h, ca, lrow128] + wlist(wv))
    ph = jnp.zeros((2, N, HIDDEN), f32)

    m = jnp.zeros((E, 2 * HH), f32)
    rpt = (-(-N // _NS) + 7) // 8 * 8
    z128 = jnp.zeros((rpt, HIDDEN), f32)

    per_w = E // _NW

    def _chunked(x, ch):
        nm = per_w // ch
        x2 = x.reshape(_NW, per_w)
        return (x2[:, :nm * ch].reshape(_NW, nm, ch), x2[:, nm * ch:])

    s64m, s64t = _chunked(src, 64)
    d64m, d64t = _chunked(dst, 64)
    pack64 = (s64m, s64t, d64m, d64t)
    vhs, vhd = _sc_gather([vh_pk], pack64)

    per_c = E // _NS
    nm2 = per_c // 128
    src2 = src.reshape(_NS, per_c)
    dst2 = dst.reshape(_NS, per_c)
    main_comb = jnp.concatenate(
        [src2[:, :nm2 * 128].reshape(_NS, nm2, 128),
         dst2[:, :nm2 * 128].reshape(_NS, nm2, 128)], axis=0)
    tail_comb = jnp.concatenate(
        [src2[:, nm2 * 128:], dst2[:, nm2 * 128:]], axis=0)

    for i in range(MP):
        we_ = wlist(_mlp_w(params["edge"][i]), [HIDDEN, HIDDEN, HIDDEN])
        wedge = [we_[0][:HIDDEN // 2], we_[0][HIDDEN // 2:],
                 we_[1][:HIDDEN // 2], we_[1][HIDDEN // 2:]] + we_[2:]
        wnode = wlist(_mlp_w(params["node"][i]), [HIDDEN, HIDDEN])
        wf = wlist(_mlp_w(params["flux"][i]), [HH, 2 * HH, HIDDEN, HIDDEN])
        w1h, w1m, w1v, w1d = wf[0], wf[1], wf[2], wf[3]
        wflux = [w1h[:HH // 2], w1h[HH // 2:], w1m,
                 w1v[:HIDDEN // 2], w1v[HIDDEN // 2:],
                 w1d[:HIDDEN // 2], w1d[HIDDEN // 2:]] + wf[4:] + [se, so]
        e_h = _tc_call(
            _edge_body, ge,
            [_rows(_BE, HIDDEN)] * 3 + [_full(a) for a in wedge],
            _rows(_BE, HIDDEN), jax.ShapeDtypeStruct((E, HIDDEN), f32),
            [vhs, vhd, e_h] + wedge)

        p = _sc_scatter(e_h, pack64, z128, N, 1)
        v_h, vh_pk, h_h = _tc_call(
            _node_body, gn,
            [_rows(_BN, HIDDEN),
             pl.BlockSpec((2, 1, _BN, HIDDEN), lambda i: (0, 0, i, 0)),
             _rows(_BN, HIDDEN),
             pl.BlockSpec((2, _BN, HIDDEN), lambda i: (0, i, 0))]
            + [_full(a) for a in wnode],
            [_rows(_BN, HIDDEN), _rows(_BN, HIDDEN), _rows(_BN, HIDDEN)],
            [jax.ShapeDtypeStruct((N, HIDDEN), f32),
             jax.ShapeDtypeStruct((N, HIDDEN), i32),
             jax.ShapeDtypeStruct((N, HIDDEN), f32)],
            [v_h, p, h_h, ph] + wnode)

        vhs, vhd = _sc_gather([vh_pk], pack64)
        m, q = _tc_call(
            _flux_body, ge,
            [_rows(_BE, HIDDEN)] * 2 + [_rows(_BE, 2 * HH)]
            + [_rows(_BE, 1)] * 3 + [_full(a) for a in wflux],
            [_rows(_BE, 2 * HH), _rows(_BE, HIDDEN)],
            [jax.ShapeDtypeStruct((E, 2 * HH), f32),
             jax.ShapeDtypeStruct((E, HIDDEN), f32)],
            [vhs, vhd, m, nx, ny, el] + wflux)

        ph = _sc_scatter_split(q, main_comb, tail_comb, z128, N)

    out = _tc_call(
        _dec_body, gn,
        [_rows(_BN, HIDDEN),
         pl.BlockSpec((2, _BN, HIDDEN), lambda i: (0, i, 0)),
         _full(lrow128), _rows(_BN, 1)],
        _rows(_BN, 1), jax.ShapeDtypeStruct((N, 1), f32),
        [h_h, ph, lrow128, ca])
    return out


# speedup vs baseline: 1.1408x; 1.1408x over previous
import functools

import jax
import jax.numpy as jnp
import numpy as np
from jax import lax
from jax.experimental import pallas as pl
from jax.experimental.pallas import tpu as pltpu
from jax.experimental.pallas import tpu_sc as plsc

HIDDEN = 128
HH = 64
MP = 5

_NC = 2
_NS = 16
_NW = _NC * _NS
_CH = 128

_BE = 4000
_BN = 2000


def _dot(x, w):
    return jnp.dot(x.astype(jnp.bfloat16), w.astype(jnp.bfloat16),
                   preferred_element_type=jnp.float32)


def _dotf(x, w):
    return jnp.dot(x, w, preferred_element_type=jnp.float32)


def _pk(a, b):
    ua = lax.bitcast_convert_type(a.astype(jnp.bfloat16),
                                  jnp.uint16).astype(jnp.uint32)
    ub = lax.bitcast_convert_type(b.astype(jnp.bfloat16),
                                  jnp.uint16).astype(jnp.uint32)
    return lax.bitcast_convert_type(ua | (ub << 16), jnp.int32)


def _unpk_lo(p):
    return lax.bitcast_convert_type(p << 16, jnp.float32)


def _unpk_hi(p):
    return lax.bitcast_convert_type(p & jnp.int32(-65536), jnp.float32)


def _mlp_tail(x1, w2, b2, w3, b3, g, bt):
    x = jnp.tanh(x1)
    x = jnp.tanh(_dot(x, w2[...]) + b2[...])
    x = _dot(x, w3[...]) + b3[...]
    mu = jnp.mean(x, axis=-1, keepdims=True)
    xc = x - mu
    var = jnp.mean(xc * xc, axis=-1, keepdims=True)
    return xc * lax.rsqrt(var + 1e-5) * g[...] + bt[...]


def _enc_e_body(e_ref, w1, b1, w2, b2, w3, b3, g, bt, out_ref):
    x1 = _dot(e_ref[...], w1[...]) + b1[...]
    out_ref[...] = _mlp_tail(x1, w2, b2, w3, b3, g, bt)


def _enc_vh_body(v_ref, h_ref, ca_ref, lrow, w1, b1, w2, b2, w3, b3, g, bt,
                 v_out, h_out, vhpk_out):
    x1 = _dot(v_ref[...], w1[...]) + b1[...]
    vv = _mlp_tail(x1, w2, b2, w3, b3, g, bt)
    v_out[...] = vv
    hh = (h_ref[...] * ca_ref[...]) * lrow[...]
    h_out[...] = hh
    hp = _pk(hh[:, :HH // 2], hh[:, HH // 2:HH])
    pkv = _pk(vv[:, :HIDDEN // 2], vv[:, HIDDEN // 2:])
    vhpk_out[...] = jnp.concatenate([pkv, hp, jnp.zeros_like(hp)], axis=-1)


def _edge_body(vhs, vhd, e, w1sa, w1sb, w1da, w1db, w1e, b1,
               w2, b2, w3, b3, g, bt, out):
    H2 = HIDDEN // 2
    ps = vhs[...][:, :H2]
    pd = vhd[...][:, :H2]
    x1 = (_dot(_unpk_lo(ps), w1sa[...]) + _dot(_unpk_hi(ps), w1sb[...])
          + _dot(_unpk_lo(pd), w1da[...]) + _dot(_unpk_hi(pd), w1db[...])
          + _dot(e[...], w1e[...]) + b1[...])
    out[...] = e[...] + _mlp_tail(x1, w2, b2, w3, b3, g, bt)


def _node_body(v, p, h_old, php, w1v, w1p, b1, w2, b2, w3, b3, g, bt,
               out, pk_out, h_out):
    pp = p[...]
    v1 = pp[0, 0] + pp[1, 0]
    x1 = _dot(v[...], w1v[...]) + _dot(v1, w1p[...]) + b1[...]
    vv = v[...] + _mlp_tail(x1, w2, b2, w3, b3, g, bt)
    out[...] = vv
    hq = php[...]
    hh = h_old[...] + hq[0] - hq[1]
    h_out[...] = hh
    pkv = _pk(vv[:, :HIDDEN // 2], vv[:, HIDDEN // 2:])
    hp = _pk(hh[:, :HH // 2], hh[:, HH // 2:HH])
    pk_out[...] = jnp.concatenate(
        [pkv, hp, jnp.zeros_like(hp)], axis=-1)


def _flux_body(vhs, vhd, m, nx, ny, el,
               w1ha, w1hb, w1m, w1va, w1vb, w1da, w1db,
               b1, w2, b2, w3, b3, g, bt, se, so,
               m_out, q_out):
    ps = vhs[...]
    pd = vhd[...]
    H2 = HIDDEN // 2
    Q = HH // 2
    vsp, vdp = ps[:, :H2], pd[:, :H2]
    hsp, hdp = ps[:, H2:H2 + Q], pd[:, H2:H2 + Q]
    hsum_lo = _unpk_lo(hsp) + _unpk_lo(hdp)
    hsum_hi = _unpk_hi(hsp) + _unpk_hi(hdp)
    x1 = (_dot(hsum_lo, w1ha[...]) + _dot(hsum_hi, w1hb[...])
          + _dot(m[...], w1m[...])
          + _dot(_unpk_lo(vsp), w1va[...]) + _dot(_unpk_hi(vsp), w1vb[...])
          + _dot(_unpk_lo(vdp), w1da[...]) + _dot(_unpk_hi(vdp), w1db[...])
          + b1[...])
    mn = m[...] + _mlp_tail(x1, w2, b2, w3, b3, g, bt)
    me = _dotf(mn, se[...])
    mo = _dotf(mn, so[...])
    m_out[...] = mn
    q = (me * nx[...] + mo * ny[...]) * el[...]
    q_out[...] = jnp.concatenate([q, jnp.zeros_like(q)], axis=-1)


def _dec_body(h, php, lrow, ca, out):
    lv = lrow[...]
    hq = php[...]
    hh = h[...] + hq[0] - hq[1]
    s = jnp.sum(lv * lv)
    out[...] = jnp.sum(hh * lv, axis=-1, keepdims=True) / s / ca[...]


def _rows(B, D):
    return pl.BlockSpec((B, D), lambda i: (i, 0))


def _full(a):
    nd = a.ndim
    return pl.BlockSpec(a.shape, lambda i: (0,) * nd)


def _tc_call(body, grid, in_specs, out_specs, out_shape, args):
    return pl.pallas_call(
        body, grid=grid, in_specs=in_specs, out_specs=out_specs,
        out_shape=out_shape)(*args)


def _sc_gather(tables, pack):
    main_src, tail_src, main_dst, tail_dst = pack
    n_main, ch = main_src.shape[1], main_src.shape[2]
    tail = tail_src.shape[1]
    per_w = n_main * ch + tail
    E = per_w * _NW
    nt = len(tables)
    ns = 2 * nt
    Ds = [int(t.shape[1]) for t in tables]
    dts = [t.dtype for t in tables]
    nb = 3
    assert n_main % nb == 0
    mesh = plsc.VectorSubcoreMesh(core_axis_name="c", subcore_axis_name="s")
    out_type = [jax.ShapeDtypeStruct((E, D), dt)
                for D, dt in zip(Ds, dts) for _ in range(2)]
    scratch = [pltpu.VMEM((n_main, ch), jnp.int32),
               pltpu.VMEM((n_main, ch), jnp.int32),
               pltpu.VMEM((tail,), jnp.int32),
               pltpu.VMEM((tail,), jnp.int32)]
    for D, dt in zip(Ds, dts):
        for _ in range(2):
            for _ in range(nb):
                scratch.append(pltpu.VMEM((ch, D), dt))
            scratch.append(pltpu.VMEM((tail, D), dt))
    nsem = ns * nb * 2 + 1
    scratch += [pltpu.SemaphoreType.DMA] * nsem

    def body(*refs):
        tab = refs[:nt]
        im = (refs[nt], refs[nt + 2])
        it = (refs[nt + 1], refs[nt + 3])
        outs = refs[nt + 4:nt + 4 + ns]
        scr = refs[nt + 4 + ns:]
        idx_all = scr[0:2]
        idx_tl = scr[2:4]
        rows = [[scr[4 + s * (nb + 1) + b] for b in range(nb)]
                for s in range(ns)]
        rtail = [scr[4 + s * (nb + 1) + nb] for s in range(ns)]
        sems = scr[4 + ns * (nb + 1):]
        sem_g = [[sems[s * nb + b] for b in range(nb)] for s in range(ns)]
        sem_w = [[sems[ns * nb + s * nb + b] for b in range(nb)]
                 for s in range(ns)]
        sem_x = sems[-1]
        wid = lax.axis_index("s") * _NC + lax.axis_index("c")
        base_w = wid * per_w

        for k in range(2):
            pltpu.sync_copy(im[k].at[wid], idx_all[k])
            pltpu.sync_copy(it[k].at[wid], idx_tl[k])

        def g_desc(s, b, j):
            t, k = s // 2, s % 2
            return pltpu.make_async_copy(
                tab[t].at[idx_all[k].at[j]], rows[s][b], sem_g[s][b])

        def w_desc(s, b, j):
            t, k = s // 2, s % 2
            return pltpu.make_async_copy(
                rows[s][b], outs[2 * t + k].at[pl.ds(base_w + j * ch, ch)],
                sem_w[s][b])

        for b in range(min(nb - 1, n_main)):
            for s in range(ns):
                g_desc(s, b, b).start()

        @pl.loop(0, n_main, step=nb)
        def _main(j0):
            for db in range(nb):
                j = j0 + db
                bp = (db - 1) % nb
                for s in range(ns):
                    @pl.when(j >= 1)
                    def _wb_done():
                        w_desc(s, bp, j - 1).wait()

                    @pl.when(j + nb - 1 < n_main)
                    def _next_g():
                        g_desc(s, bp, j + nb - 1).start()

                    g_desc(s, db, j).wait()
                    w_desc(s, db, j).start()

        for s in range(ns):
            w_desc(s, (n_main - 1) % nb, n_main - 1).wait()

        if tail:
            for s in range(ns):
                t, k = s // 2, s % 2
                pltpu.make_async_copy(
                    tab[t].at[idx_tl[k]], rtail[s], sem_x).start()
            for s in range(ns):
                t, k = s // 2, s % 2
                pltpu.make_async_copy(
                    tab[t].at[idx_tl[k]], rtail[s], sem_x).wait()
                pltpu.sync_copy(
                    rtail[s],
                    outs[2 * t + k].at[pl.ds(base_w + n_main * ch, tail)])

    f = pl.kernel(body, out_type=out_type, mesh=mesh, scratch_types=scratch)
    return f(*tables, main_src, tail_src, main_dst, tail_dst)


def _sc_scatter_split(data, main_comb, tail_comb, zrows, n_nodes):
    E, D = data.shape
    n_main, ch = main_comb.shape[1], main_comb.shape[2]
    tail = tail_comb.shape[1]
    per_c = n_main * ch + tail
    rpt = (-(-n_nodes // _NS) + 7) // 8 * 8
    last = n_nodes - (_NS - 1) * rpt
    mesh = plsc.VectorSubcoreMesh(core_axis_name="c", subcore_axis_name="s")
    out_type = jax.ShapeDtypeStruct((2, n_nodes, D), jnp.float32)
    nb = 2
    assert n_main % nb == 0
    scratch = ([pltpu.VMEM_SHARED((n_nodes, D), jnp.float32)]
               + [pltpu.VMEM((n_main, ch), jnp.int32)]
               + [pltpu.VMEM((tail,), jnp.int32)]
               + [pltpu.VMEM((ch, D), jnp.float32)] * nb
               + [pltpu.VMEM((tail, D), jnp.float32)]
               + [pltpu.SemaphoreType.DMA] * (2 * nb + 1))

    def body(d_hbm, imc, itc, zr, out, *scr):
        accum = scr[0]
        idx_all = scr[1]
        idx_tl = scr[2]
        rows = scr[3:3 + nb]
        rtail = scr[3 + nb]
        sems = scr[4 + nb:]
        sem_l = sems[0:nb]
        sem_s = sems[nb:2 * nb]
        sem_x = sems[-1]
        cid = lax.axis_index("c")
        sid = lax.axis_index("s")
        base_w = sid * per_c

        @pl.when(sid < _NS - 1)
        def _zmain():
            pltpu.sync_copy(zr, accum.at[pl.ds(sid * rpt, rpt)])

        @pl.when(sid == _NS - 1)
        def _zlast():
            pltpu.sync_copy(zr.at[pl.ds(0, last)],
                            accum.at[pl.ds((_NS - 1) * rpt, last)])

        pltpu.sync_copy(imc.at[cid * _NS + sid], idx_all)
        pltpu.sync_copy(itc.at[cid * _NS + sid], idx_tl)
        plsc.subcore_barrier()

        def l_desc(b, j):
            return pltpu.make_async_copy(
                d_hbm.at[pl.ds(base_w + j * ch, ch)], rows[b], sem_l[b])

        def s_desc(b, j):
            return pltpu.make_async_copy(
                rows[b], accum.at[idx_all.at[j]], sem_s[b])

        for b in range(min(nb - 1, n_main)):
            l_desc(b, b).start()

        @pl.loop(0, n_main, step=nb)
        def _main(j0):
            for db in range(nb):
                j = j0 + db
                bp = (db - 1) % nb

                @pl.when(j >= 1)
                def _prev_done():
                    s_desc(bp, j - 1).wait()

                @pl.when(j + nb - 1 < n_main)
                def _next_l():
                    l_desc(bp, j + nb - 1).start()

                l_desc(db, j).wait()
                pltpu.async_copy(rows[db], accum.at[idx_all.at[j]],
                                 sem_s[db], add=True)

        s_desc((n_main - 1) % nb, n_main - 1).wait()

        if tail:
            pltpu.make_async_copy(
                d_hbm.at[pl.ds(base_w + n_main * ch, tail)], rtail,
                sem_x).start()
            pltpu.make_async_copy(
                d_hbm.at[pl.ds(base_w + n_main * ch, tail)], rtail,
                sem_x).wait()
            pltpu.sync_copy(rtail, accum.at[idx_tl], add=True)

        plsc.subcore_barrier()

        @pl.when(sid < _NS - 1)
        def _omain():
            pltpu.sync_copy(accum.at[pl.ds(sid * rpt, rpt)],
                            out.at[cid, pl.ds(sid * rpt, rpt)])

        @pl.when(sid == _NS - 1)
        def _olast():
            pltpu.sync_copy(accum.at[pl.ds((_NS - 1) * rpt, last)],
                            out.at[cid, pl.ds((_NS - 1) * rpt, last)])

    f = pl.kernel(body, out_type=out_type, mesh=mesh, scratch_types=scratch)
    return f(data, main_comb, tail_comb, zrows)


def _sc_scatter(data, pack, zrows, n_nodes, na):
    main_src, tail_src, main_dst, tail_dst = pack
    E, D = data.shape
    n_main, ch = main_src.shape[1], main_src.shape[2]
    tail = tail_src.shape[1]
    per_w = n_main * ch + tail
    rpt = (-(-n_nodes // _NS) + 7) // 8 * 8
    last = n_nodes - (_NS - 1) * rpt
    mesh = plsc.VectorSubcoreMesh(core_axis_name="c", subcore_axis_name="s")
    out_type = jax.ShapeDtypeStruct((2, na, n_nodes, D), jnp.float32)
    nb = 2
    assert n_main % nb == 0
    scratch = ([pltpu.VMEM_SHARED((n_nodes, D), jnp.float32)] * na
               + [pltpu.VMEM((n_main, ch), jnp.int32)] * 2
               + [pltpu.VMEM((tail,), jnp.int32)] * 2
               + [pltpu.VMEM((ch, D), jnp.float32)] * nb
               + [pltpu.VMEM((tail, D), jnp.float32)]
               + [pltpu.SemaphoreType.DMA] * (3 * nb + 1))

    def body(d_hbm, ims, its, imd, itd, zr, out, *scr):
        accums = scr[:na]
        idx_all = scr[na:na + 2]
        idx_tl = scr[na + 2:na + 4]
        rows = scr[na + 4:na + 4 + nb]
        rtail = scr[na + 4 + nb]
        sems = scr[na + 5 + nb:]
        sem_l = sems[0:nb]
        sem_s = [sems[nb + k * nb:nb + (k + 1) * nb]
                 for k in range(2)]
        sem_x = sems[-1]
        cid = lax.axis_index("c")
        sid = lax.axis_index("s")
        wid = sid * _NC + cid
        base_w = wid * per_w
        acc = [accums[0], accums[na - 1]]

        for a in range(na):
            @pl.when(sid < _NS - 1)
            def _zmain():
                pltpu.sync_copy(zr, accums[a].at[pl.ds(sid * rpt, rpt)])

            @pl.when(sid == _NS - 1)
            def _zlast():
                pltpu.sync_copy(zr.at[pl.ds(0, last)],
                                accums[a].at[pl.ds((_NS - 1) * rpt, last)])

        pltpu.sync_copy(ims.at[wid], idx_all[0])
        pltpu.sync_copy(imd.at[wid], idx_all[1])
        pltpu.sync_copy(its.at[wid], idx_tl[0])
        pltpu.sync_copy(itd.at[wid], idx_tl[1])
        plsc.subcore_barrier()

        def l_desc(b, j):
            return pltpu.make_async_copy(
                d_hbm.at[pl.ds(base_w + j * ch, ch)], rows[b], sem_l[b])

        def s_desc(k, b, j):
            return pltpu.make_async_copy(
                rows[b], acc[k].at[idx_all[k].at[j]], sem_s[k][b])

        for b in range(min(nb - 1, n_main)):
            l_desc(b, b).start()

        @pl.loop(0, n_main, step=nb)
        def _main(j0):
            for db in range(nb):
                j = j0 + db
                bp = (db - 1) % nb

                @pl.when(j >= 1)
                def _prev_done():
                    for k in range(2):
                        s_desc(k, bp, j - 1).wait()

                @pl.when(j + nb - 1 < n_main)
                def _next_l():
                    l_desc(bp, j + nb - 1).start()

                l_desc(db, j).wait()
                for k in range(2):
                    pltpu.async_copy(rows[db], acc[k].at[idx_all[k].at[j]],
                                     sem_s[k][db], add=True)

        for k in range(2):
            s_desc(k, (n_main - 1) % nb, n_main - 1).wait()

        if tail:
            pltpu.make_async_copy(
                d_hbm.at[pl.ds(base_w + n_main * ch, tail)], rtail,
                sem_x).start()
            pltpu.make_async_copy(
                d_hbm.at[pl.ds(base_w + n_main * ch, tail)], rtail,
                sem_x).wait()
            for k in range(2):
                pltpu.sync_copy(rtail, acc[k].at[idx_tl[k]], add=True)

        plsc.subcore_barrier()
        for a in range(na):
            @pl.when(sid < _NS - 1)
            def _omain():
                pltpu.sync_copy(accums[a].at[pl.ds(sid * rpt, rpt)],
                                out.at[cid, a, pl.ds(sid * rpt, rpt)])

            @pl.when(sid == _NS - 1)
            def _olast():
                pltpu.sync_copy(accums[a].at[pl.ds((_NS - 1) * rpt, last)],
                                out.at[cid, a, pl.ds((_NS - 1) * rpt, last)])

    f = pl.kernel(body, out_type=out_type, mesh=mesh, scratch_types=scratch)
    return f(data, main_src, tail_src, main_dst, tail_dst, zrows)


def _mlp_w(p):
    return dict(
        w1=p["l1"]["w"].T, b1=p["l1"]["b"].reshape(1, -1),
        w2=p["l2"]["w"].T, b2=p["l2"]["b"].reshape(1, -1),
        w3=p["l3"]["w"].T, b3=p["l3"]["b"].reshape(1, -1),
        g=p["g"].reshape(1, -1), bt=p["bt"].reshape(1, -1))


def kernel(h, e, v, edges, cell_areas, edge_lens, edge_normals, params):
    N = h.shape[0]
    E = e.shape[0]
    f32 = jnp.float32
    src = edges[:, 0].astype(jnp.int32)
    dst = edges[:, 1].astype(jnp.int32)
    ca = cell_areas.reshape(N, 1).astype(f32)
    nx = edge_normals[:, 0:1]
    ny = edge_normals[:, 1:2]
    el = edge_lens.reshape(E, 1)
    lrow = params["L"].reshape(1, HH)
    lrow128 = jnp.concatenate([lrow, jnp.zeros((1, HH), f32)], axis=-1)

    ge = (E // _BE,)
    gn = (N // _BN,)

    se_np = np.zeros((2 * HH, HH), np.float32)
    so_np = np.zeros((2 * HH, HH), np.float32)
    se_np[2 * np.arange(HH), np.arange(HH)] = 1.0
    so_np[2 * np.arange(HH) + 1, np.arange(HH)] = 1.0
    se = jnp.asarray(se_np)
    so = jnp.asarray(so_np)

    we = _mlp_w(params["e_enc"])
    wv = _mlp_w(params["v_enc"])

    def wlist(w, first_splits=None):
        if first_splits is None:
            firsts = [w["w1"]]
        else:
            firsts = []
            o = 0
            for sz in first_splits:
                firsts.append(w["w1"][o:o + sz])
                o += sz
        return firsts + [w["b1"], w["w2"], w["b2"], w["w3"], w["b3"],
                         w["g"], w["bt"]]

    e_h = _tc_call(
        _enc_e_body, ge,
        [_rows(_BE, e.shape[1])] + [_full(a) for a in wlist(we)],
        _rows(_BE, HIDDEN), jax.ShapeDtypeStruct((E, HIDDEN), f32),
        [e] + wlist(we))

    i32 = jnp.int32
    v_h, h_h, vh_pk = _tc_call(
        _enc_vh_body, gn,
        [_rows(_BN, v.shape[1]), _rows(_BN, 1), _rows(_BN, 1), _full(lrow128)]
        + [_full(a) for a in wlist(wv)],
        [_rows(_BN, HIDDEN), _rows(_BN, HIDDEN), _rows(_BN, HIDDEN)],
        [jax.ShapeDtypeStruct((N, HIDDEN), f32),
         jax.ShapeDtypeStruct((N, HIDDEN), f32),
         jax.ShapeDtypeStruct((N, HIDDEN), i32)],
        [v, h, ca, lrow128] + wlist(wv))
    ph = jnp.zeros((2, N, HIDDEN), f32)

    m = jnp.zeros((E, 2 * HH), f32)
    rpt = (-(-N // _NS) + 7) // 8 * 8
    z128 = jnp.zeros((rpt, HIDDEN), f32)

    per_w = E // _NW

    def _chunked(x, ch):
        nm = per_w // ch
        x2 = x.reshape(_NW, per_w)
        return (x2[:, :nm * ch].reshape(_NW, nm, ch), x2[:, nm * ch:])

    s64m, s64t = _chunked(src, 64)
    d64m, d64t = _chunked(dst, 64)
    pack64 = (s64m, s64t, d64m, d64t)
    vhs, vhd = _sc_gather([vh_pk], pack64)

    per_c = E // _NS
    nm2 = per_c // 128
    src2 = src.reshape(_NS, per_c)
    dst2 = dst.reshape(_NS, per_c)
    main_comb = jnp.concatenate(
        [src2[:, :nm2 * 128].reshape(_NS, nm2, 128),
         dst2[:, :nm2 * 128].reshape(_NS, nm2, 128)], axis=0)
    tail_comb = jnp.concatenate(
        [src2[:, nm2 * 128:], dst2[:, nm2 * 128:]], axis=0)

    for i in range(MP):
        we_ = wlist(_mlp_w(params["edge"][i]), [HIDDEN, HIDDEN, HIDDEN])
        wedge = [we_[0][:HIDDEN // 2], we_[0][HIDDEN // 2:],
                 we_[1][:HIDDEN // 2], we_[1][HIDDEN // 2:]] + we_[2:]
        wnode = wlist(_mlp_w(params["node"][i]), [HIDDEN, HIDDEN])
        wf = wlist(_mlp_w(params["flux"][i]), [HH, 2 * HH, HIDDEN, HIDDEN])
        w1h, w1m, w1v, w1d = wf[0], wf[1], wf[2], wf[3]
        wflux = [w1h[:HH // 2], w1h[HH // 2:], w1m,
                 w1v[:HIDDEN // 2], w1v[HIDDEN // 2:],
                 w1d[:HIDDEN // 2], w1d[HIDDEN // 2:]] + wf[4:] + [se, so]
        e_h = _tc_call(
            _edge_body, ge,
            [_rows(_BE, HIDDEN)] * 3 + [_full(a) for a in wedge],
            _rows(_BE, HIDDEN), jax.ShapeDtypeStruct((E, HIDDEN), f32),
            [vhs, vhd, e_h] + wedge)

        p = _sc_scatter(e_h, pack64, z128, N, 1)
        v_h, vh_pk, h_h = _tc_call(
            _node_body, gn,
            [_rows(_BN, HIDDEN),
             pl.BlockSpec((2, 1, _BN, HIDDEN), lambda i: (0, 0, i, 0)),
             _rows(_BN, HIDDEN),
             pl.BlockSpec((2, _BN, HIDDEN), lambda i: (0, i, 0))]
            + [_full(a) for a in wnode],
            [_rows(_BN, HIDDEN), _rows(_BN, HIDDEN), _rows(_BN, HIDDEN)],
            [jax.ShapeDtypeStruct((N, HIDDEN), f32),
             jax.ShapeDtypeStruct((N, HIDDEN), i32),
             jax.ShapeDtypeStruct((N, HIDDEN), f32)],
            [v_h, p, h_h, ph] + wnode)

        vhs, vhd = _sc_gather([vh_pk], pack64)
        m, q = _tc_call(
            _flux_body, ge,
            [_rows(_BE, HIDDEN)] * 2 + [_rows(_BE, 2 * HH)]
            + [_rows(_BE, 1)] * 3 + [_full(a) for a in wflux],
            [_rows(_BE, 2 * HH), _rows(_BE, HIDDEN)],
            [jax.ShapeDtypeStruct((E, 2 * HH), f32),
             jax.ShapeDtypeStruct((E, HIDDEN), f32)],
            [vhs, vhd, m, nx, ny, el] + wflux)

        ph = _sc_scatter_split(q, main_comb, tail_comb, z128, N)

    out = _tc_call(
        _dec_body, gn,
        [_rows(_BN, HIDDEN),
         pl.BlockSpec((2, _BN, HIDDEN), lambda i: (0, i, 0)),
         _full(lrow128), _rows(_BN, 1)],
        _rows(_BN, 1), jax.ShapeDtypeStruct((N, 1), f32),
        [h_h, ph, lrow128, ca])
    return out
